# Initial kernel scaffold; baseline (speedup 1.0000x reference)
import jax
import jax.numpy as jnp
from jax.experimental import pallas as pl


def kernel(x):
    raise NotImplementedError("write your pallas kernel here")


# speedup vs baseline: 1.0436x; 1.0436x over previous
import jax
import jax.numpy as jnp
from jax.experimental import pallas as pl

_SHAPE = (640, 480)
_GROUP_NUM = 8
_PATCH = (16, 16)
_H = _SHAPE[1] - 1
_W = _SHAPE[0] - 1
_TIME_DIV = _GROUP_NUM // 2
_PH = (_H + 1) // _PATCH[0]
_PW = (_W + 1) // _PATCH[1]
_TOKEN_NUM = _PH * _PW
_PATCH_SIZE = _PATCH[0] * _PATCH[1]
_TOTAL = _TIME_DIV * 2 * _PATCH_SIZE * _TOKEN_NUM
_B = 0.0001


def _index_kernel(t_ref, xs_ref, ys_ref, p_ref, dt_ref, s_ref,
                  l_ref, w_ref, wt_ref):
    t = t_ref[...]
    xs = xs_ref[...]
    ys = ys_ref[...]
    p = p_ref[...]
    dt = dt_ref[...]
    t0 = s_ref[0]
    wt_scale = s_ref[1]

    w_ref[...] = (p != jnp.float32(2.0)).astype(jnp.float32)
    wt_ref[...] = (t - t0) * wt_scale

    cx = jnp.float32(_W / _PW + _B)
    cy = jnp.float32(_H / _PH + _B)
    cxm = jnp.float32(_W / _PW + 0.0001)
    pos = jnp.floor(xs / cx) + jnp.floor(ys / cy) * jnp.float32(_PW)
    tok = jnp.floor(xs % cxm) + jnp.floor(ys % cy) * jnp.float32((_W + 1) // _PW)

    dt_i = jnp.clip(dt, 0, _TIME_DIV)
    p_i = jnp.clip(p.astype(jnp.int32), 0, 2)
    tok_i = jnp.clip(tok.astype(jnp.int32), 0, _PATCH_SIZE)
    pos_i = jnp.clip(pos.astype(jnp.int32), 0, _TOKEN_NUM)

    l_ref[...] = (dt_i * (2 * _PATCH_SIZE * _TOKEN_NUM)
                  + p_i * (_PATCH_SIZE * _TOKEN_NUM)
                  + tok_i * _TOKEN_NUM
                  + pos_i)


def kernel(x):
    x = x.reshape(-1, 4)
    n = x.shape[0]
    t = x[:, 0]
    xs = x[:, 1]
    ys = x[:, 2]
    p = x[:, 3]

    t0 = t[0]
    tN = t[n - 1]
    wt_scale = 1.0 / (tN - t0 + 0.0001)
    scalars = jnp.stack([t0, wt_scale]).astype(jnp.float32)

    t_d = t.astype(jnp.float64)
    dt = jnp.floor(_TIME_DIV * (t_d - t_d[0]) / (t_d[n - 1] - t_d[0] + 1.0)).astype(jnp.int32)

    chunk = 131072
    chunks = -(-n // chunk)
    n_pad = chunks * chunk
    pad = n_pad - n
    if pad:
        t, xs, ys, p, dt = (jnp.pad(a, (0, pad)) for a in (t, xs, ys, p, dt))

    ev_spec = pl.BlockSpec((chunk,), lambda i: (jnp.int32(i),))
    l, w, wt = pl.pallas_call(
        _index_kernel,
        grid=(chunks,),
        in_specs=[ev_spec, ev_spec, ev_spec, ev_spec, ev_spec,
                  pl.BlockSpec((2,), lambda i: (jnp.int32(0),))],
        out_specs=[ev_spec, ev_spec, ev_spec],
        out_shape=[
            jax.ShapeDtypeStruct((n_pad,), jnp.int32),
            jax.ShapeDtypeStruct((n_pad,), jnp.float32),
            jax.ShapeDtypeStruct((n_pad,), jnp.float32),
        ],
    )(t, xs, ys, p, dt, scalars)
    l, w, wt = l[:n], w[:n], wt[:n]

    hist = jnp.zeros((_TOTAL,), jnp.float32).at[l].add(w)
    hist2 = jnp.zeros((_TOTAL,), jnp.float32).at[l].add(wt)
    hist = hist.reshape(_TIME_DIV, 2, _PATCH_SIZE, _TOKEN_NUM)
    hist2 = hist2.reshape(_TIME_DIV, 2, _PATCH_SIZE, _TOKEN_NUM)
    y = jnp.stack([hist, hist2], axis=2).astype(jnp.float32)
    return y.reshape(1, -1, _PH, _PW)
